# baseline (speedup 1.0000x reference)
import jax
import jax.numpy as jnp
from jax import lax
from jax.experimental import pallas as pl
from jax.experimental.pallas import tpu as pltpu
from jax.experimental.pallas import tpu_sc as plsc

N = 16777216
NUM_WORKERS = 32
PER_WORKER = N // NUM_WORKERS
CHUNK = 16384
NCHUNKS = PER_WORKER // CHUNK
BINS = 256
DUMP = 512
REGION = 528
SHARED_N = 16 * REGION
NVST = 8
NSTR = 8
STR_WORDS = 64 * NSTR * 16


def _hist_body(x_hbm, y_hbm, out_hbm, buf0, buf1, hist, idx0, idx1, idx2,
               idx3, ones2d, zbuf, local, shared, sem0, sem1, ssem0, ssem1,
               ssem2, ssem3):
    core = lax.axis_index("c")
    sub = lax.axis_index("s")
    wid = sub * 2 + core
    base = wid * PER_WORKER
    sems = (sem0, sem1)
    ssems = (ssem0, ssem1, ssem2, ssem3)
    bufs_all = (buf0, buf1)
    idxbufs = (idx0, idx1, idx2, idx3)

    zeros16 = jnp.zeros((16,), jnp.float32)
    ones16 = jnp.ones((16,), jnp.float32)
    lane = lax.broadcasted_iota(jnp.int32, (16,), 0)

    def zb(i, _):
        hist[pl.ds(i * 16, 16)] = zeros16
        return 0

    lax.fori_loop(0, 512, zb, 0)

    def of(r, _):
        ones2d[pl.ds(r * 16, 16)] = ones16
        return 0

    lax.fori_loop(0, STR_WORDS // 16, of, 0)

    @pl.when(sub == 0)
    def _zero_shared():
        def zs(i, _):
            zbuf[pl.ds(i * 16, 16)] = zeros16
            return 0

        lax.fori_loop(0, SHARED_N // 16, zs, 0)
        pltpu.sync_copy(zbuf, shared)

    plsc.subcore_barrier()

    for which, src in enumerate((x_hbm, y_hbm)):
        lanewhich = lane + which * 4096
        woff = which * BINS + sub * REGION
        dump = sub * REGION + DUMP

        pltpu.async_copy(src.at[pl.ds(base, CHUNK)], buf0, sem0)

        def obody(g, _, src=src, lanewhich=lanewhich, woff=woff, dump=dump):
            for s2 in range(4):
                c = g * 4 + s2
                db = s2 % 2
                pltpu.make_async_copy(
                    src.at[pl.ds(base, CHUNK)], bufs_all[db], sems[db]
                ).wait()

                @pl.when(c + 1 < NCHUNKS)
                def _start(db=db, c=c, src=src):
                    pltpu.async_copy(
                        src.at[pl.ds(base + (c + 1) * CHUNK, CHUNK)],
                        bufs_all[db ^ 1],
                        sems[db ^ 1],
                    )

                @pl.when(g >= 1)
                def _wait_stream(s2=s2):
                    pltpu.make_async_copy(
                        ones2d, shared.at[idxbufs[s2]], ssems[s2]
                    ).wait()

                bufs = bufs_all[db]
                ib = idxbufs[s2]

                def vbody(i, _, bufs=bufs, ib=ib, lanewhich=lanewhich,
                          woff=woff, dump=dump):
                    b0 = i * 16
                    for k in range(NVST):
                        v = bufs[pl.ds((b0 + k) * 16, 16)]
                        t = (v + 4.0) * 32.0
                        u = t.astype(jnp.int32)
                        mask = (plsc.bitcast(t, jnp.uint32)
                                <= jnp.uint32(0x43800000))
                        idx = jnp.minimum(u, BINS - 1)
                        flat = (idx << 4) + lanewhich
                        plsc.addupdate_scatter(hist, [flat], ones16,
                                               mask=mask)
                    for k in range(NSTR):
                        v = bufs[pl.ds((b0 + NVST + k) * 16, 16)]
                        t = (v + 4.0) * 32.0
                        u = t.astype(jnp.int32)
                        mask = (plsc.bitcast(t, jnp.uint32)
                                <= jnp.uint32(0x43800000))
                        idx = jnp.minimum(u, BINS - 1)
                        sidx = jnp.where(mask, idx + woff, dump)
                        ib[pl.ds(i * (NSTR * 16) + k * 16, 16)] = sidx
                    return 0

                lax.fori_loop(0, 64, vbody, 0)

                pltpu.async_copy(ones2d, shared.at[ib], ssems[s2], add=True)
            return 0

        lax.fori_loop(0, NCHUNKS // 4, obody, 0)

        for p in range(4):
            pltpu.make_async_copy(
                ones2d, shared.at[idxbufs[p]], ssems[p]
            ).wait()

    for which in range(2):
        base_vec = lane * 16 + which * 4096
        for j in range(BINS // 16):
            acc = zeros16
            for l in range(16):
                acc = acc + plsc.load_gather(hist, [base_vec + (j * 256 + l)])
            local[pl.ds(which * 256 + j * 16, 16)] = acc

    plsc.subcore_barrier()

    @pl.when(sub == 0)
    def _merge():
        pltpu.sync_copy(shared, zbuf)
        for r in range(16):
            for j in range(512 // 16):
                lv = (local[pl.ds(j * 16, 16)]
                      + zbuf[pl.ds(r * REGION + j * 16, 16)])
                local[pl.ds(j * 16, 16)] = lv

    pltpu.sync_copy(local, out_hbm.at[wid])


def _make_hist_kernel():
    mesh = plsc.VectorSubcoreMesh(core_axis_name="c", subcore_axis_name="s")
    return pl.kernel(
        _hist_body,
        mesh=mesh,
        compiler_params=pltpu.CompilerParams(needs_layout_passes=False),
        out_type=jax.ShapeDtypeStruct((NUM_WORKERS, 512), jnp.float32),
        scratch_types=[
            pltpu.VMEM((CHUNK,), jnp.float32),
            pltpu.VMEM((CHUNK,), jnp.float32),
            pltpu.VMEM((8192,), jnp.float32),
            pltpu.VMEM((STR_WORDS,), jnp.int32),
            pltpu.VMEM((STR_WORDS,), jnp.int32),
            pltpu.VMEM((STR_WORDS,), jnp.int32),
            pltpu.VMEM((STR_WORDS,), jnp.int32),
            pltpu.VMEM((STR_WORDS,), jnp.float32),
            pltpu.VMEM((SHARED_N,), jnp.float32),
            pltpu.VMEM((512,), jnp.float32),
            pltpu.VMEM_SHARED((SHARED_N,), jnp.float32),
            pltpu.SemaphoreType.DMA,
            pltpu.SemaphoreType.DMA,
            pltpu.SemaphoreType.DMA,
            pltpu.SemaphoreType.DMA,
            pltpu.SemaphoreType.DMA,
            pltpu.SemaphoreType.DMA,
        ],
    )


def _mi_body(p_ref, o_ref):
    p = p_ref[...]
    s = jnp.sum(p, axis=0, keepdims=True)
    hx = s[:, :BINS]
    hy = s[:, BINS:]
    sx = jnp.sum(hx)
    sy = jnp.sum(hy)
    jp = (hx + hy) / (sx + sy)
    px = hx / sx
    py = hy / sy
    ljp = jnp.log(jp)
    lpx = jnp.log(px)
    lpy = jnp.log(py)
    a = jp * (ljp - lpy)
    mi = float(BINS) * jnp.sum(a) - jnp.sum(lpx) * jnp.sum(jp)
    o_ref[...] = jnp.reshape(-mi, (1, 1))


def _mi_reduce(partials):
    return pl.pallas_call(
        _mi_body,
        out_shape=jax.ShapeDtypeStruct((1, 1), jnp.float32),
    )(partials)


@jax.jit
def kernel(x, y):
    partials = _make_hist_kernel()(x, y)
    out = _mi_reduce(partials)
    return out[0, 0]


# speedup vs baseline: 1.2633x; 1.2633x over previous
import jax
import jax.numpy as jnp
from jax import lax
from jax.experimental import pallas as pl
from jax.experimental.pallas import tpu as pltpu
from jax.experimental.pallas import tpu_sc as plsc

N = 16777216
NUM_WORKERS = 32
CHUNK = 16384
NCHUNKS = 20
PER_WORKER = NCHUNKS * CHUNK
A_SC = NUM_WORKERS * PER_WORKER
TC_RB = 512
TC_BLOCK0 = A_SC // (TC_RB * 128)
TC_NSTEPS = (N - A_SC) // (TC_RB * 128)
BINS = 256
DUMP = 512
REGION = 528
SHARED_N = 16 * REGION
NVST = 8
NSTR = 8
STR_WORDS = 64 * NSTR * 16


def _hist_body(x_hbm, y_hbm, out_hbm, buf0, buf1, hist, idx0, idx1, idx2,
               idx3, ones2d, zbuf, local, shared, sem0, sem1, ssem0, ssem1,
               ssem2, ssem3):
    core = lax.axis_index("c")
    sub = lax.axis_index("s")
    wid = sub * 2 + core
    base = wid * PER_WORKER
    sems = (sem0, sem1)
    ssems = (ssem0, ssem1, ssem2, ssem3)
    bufs_all = (buf0, buf1)
    idxbufs = (idx0, idx1, idx2, idx3)

    zeros16 = jnp.zeros((16,), jnp.float32)
    ones16 = jnp.ones((16,), jnp.float32)
    lane = lax.broadcasted_iota(jnp.int32, (16,), 0)

    def zb(i, _):
        hist[pl.ds(i * 16, 16)] = zeros16
        return 0

    lax.fori_loop(0, 512, zb, 0)

    def of(r, _):
        ones2d[pl.ds(r * 16, 16)] = ones16
        return 0

    lax.fori_loop(0, STR_WORDS // 16, of, 0)

    @pl.when(sub == 0)
    def _zero_shared():
        def zs(i, _):
            zbuf[pl.ds(i * 16, 16)] = zeros16
            return 0

        lax.fori_loop(0, SHARED_N // 16, zs, 0)
        pltpu.sync_copy(zbuf, shared)

    plsc.subcore_barrier()

    for which, src in enumerate((x_hbm, y_hbm)):
        lanewhich = lane + which * 4096
        woff = which * BINS + sub * REGION
        dump = sub * REGION + DUMP

        pltpu.async_copy(src.at[pl.ds(base, CHUNK)], buf0, sem0)

        def obody(g, _, src=src, lanewhich=lanewhich, woff=woff, dump=dump):
            for s2 in range(4):
                c = g * 4 + s2
                db = s2 % 2
                pltpu.make_async_copy(
                    src.at[pl.ds(base, CHUNK)], bufs_all[db], sems[db]
                ).wait()

                @pl.when(c + 1 < NCHUNKS)
                def _start(db=db, c=c, src=src):
                    pltpu.async_copy(
                        src.at[pl.ds(base + (c + 1) * CHUNK, CHUNK)],
                        bufs_all[db ^ 1],
                        sems[db ^ 1],
                    )

                @pl.when(g >= 1)
                def _wait_stream(s2=s2):
                    pltpu.make_async_copy(
                        ones2d, shared.at[idxbufs[s2]], ssems[s2]
                    ).wait()

                bufs = bufs_all[db]
                ib = idxbufs[s2]

                def vbody(i, _, bufs=bufs, ib=ib, lanewhich=lanewhich,
                          woff=woff, dump=dump):
                    b0 = i * 16
                    for k in range(NVST):
                        v = bufs[pl.ds((b0 + k) * 16, 16)]
                        t = (v + 4.0) * 32.0
                        u = t.astype(jnp.int32)
                        mask = (plsc.bitcast(t, jnp.uint32)
                                <= jnp.uint32(0x43800000))
                        idx = jnp.minimum(u, BINS - 1)
                        flat = (idx << 4) + lanewhich
                        plsc.addupdate_scatter(hist, [flat], ones16,
                                               mask=mask)
                    for k in range(NSTR):
                        v = bufs[pl.ds((b0 + NVST + k) * 16, 16)]
                        t = (v + 4.0) * 32.0
                        u = t.astype(jnp.int32)
                        mask = (plsc.bitcast(t, jnp.uint32)
                                <= jnp.uint32(0x43800000))
                        idx = jnp.minimum(u, BINS - 1)
                        sidx = jnp.where(mask, idx + woff, dump)
                        ib[pl.ds(i * (NSTR * 16) + k * 16, 16)] = sidx
                    return 0

                lax.fori_loop(0, 64, vbody, 0)

                pltpu.async_copy(ones2d, shared.at[ib], ssems[s2], add=True)
            return 0

        lax.fori_loop(0, NCHUNKS // 4, obody, 0)

        for p in range(4):
            pltpu.make_async_copy(
                ones2d, shared.at[idxbufs[p]], ssems[p]
            ).wait()

    for which in range(2):
        base_vec = lane * 16 + which * 4096
        for j in range(BINS // 16):
            acc = zeros16
            for l in range(16):
                acc = acc + plsc.load_gather(hist, [base_vec + (j * 256 + l)])
            local[pl.ds(which * 256 + j * 16, 16)] = acc

    plsc.subcore_barrier()

    @pl.when(sub == 0)
    def _merge():
        pltpu.sync_copy(shared, zbuf)
        for r in range(16):
            for j in range(512 // 16):
                lv = (local[pl.ds(j * 16, 16)]
                      + zbuf[pl.ds(r * REGION + j * 16, 16)])
                local[pl.ds(j * 16, 16)] = lv

    pltpu.sync_copy(local, out_hbm.at[wid])


def _make_hist_kernel():
    mesh = plsc.VectorSubcoreMesh(core_axis_name="c", subcore_axis_name="s")
    return pl.kernel(
        _hist_body,
        mesh=mesh,
        compiler_params=pltpu.CompilerParams(needs_layout_passes=False),
        out_type=jax.ShapeDtypeStruct((NUM_WORKERS, 512), jnp.float32),
        scratch_types=[
            pltpu.VMEM((CHUNK,), jnp.float32),
            pltpu.VMEM((CHUNK,), jnp.float32),
            pltpu.VMEM((8192,), jnp.float32),
            pltpu.VMEM((STR_WORDS,), jnp.int32),
            pltpu.VMEM((STR_WORDS,), jnp.int32),
            pltpu.VMEM((STR_WORDS,), jnp.int32),
            pltpu.VMEM((STR_WORDS,), jnp.int32),
            pltpu.VMEM((STR_WORDS,), jnp.float32),
            pltpu.VMEM((SHARED_N,), jnp.float32),
            pltpu.VMEM((512,), jnp.float32),
            pltpu.VMEM_SHARED((SHARED_N,), jnp.float32),
            pltpu.SemaphoreType.DMA,
            pltpu.SemaphoreType.DMA,
            pltpu.SemaphoreType.DMA,
            pltpu.SemaphoreType.DMA,
            pltpu.SemaphoreType.DMA,
            pltpu.SemaphoreType.DMA,
        ],
    )


def _tc_hist_body(x_ref, o_ref, h_ref):
    step = pl.program_id(0)

    @pl.when(step == 0)
    def _init():
        o_ref[...] = jnp.zeros((128, 128), jnp.float32)

    v = x_ref[...]
    t = (v + 4.0) * 32.0
    u = t.astype(jnp.int32)
    valid = lax.bitcast_convert_type(t, jnp.uint32) <= jnp.uint32(0x43800000)
    idx = jnp.minimum(u, 255)
    hi = jnp.where(valid, idx >> 4, 16)
    lo = idx & 15
    hibf = hi.astype(jnp.bfloat16)
    lobf = lo.astype(jnp.bfloat16)

    i128 = lax.broadcasted_iota(jnp.int32, (128, 1), 0)
    patc = (i128 & 15).astype(jnp.float32)
    rep = ((lax.broadcasted_iota(jnp.int32, (128, 8), 0) >> 4)
           == lax.broadcasted_iota(jnp.int32, (128, 8), 1)
           ).astype(jnp.bfloat16)

    accs = [jnp.zeros((128, 128), jnp.float32) for _ in range(8)]
    for r8 in range(TC_RB // 8):
        h8 = hibf[r8 * 8:(r8 + 1) * 8, :]
        l8 = lobf[r8 * 8:(r8 + 1) * 8, :]
        eh = jnp.dot(rep, h8, preferred_element_type=jnp.float32)
        el = jnp.dot(rep, l8, preferred_element_type=jnp.float32)
        oh = (eh == patc).astype(jnp.bfloat16)
        ol = (el == patc).astype(jnp.bfloat16)
        accs[r8 % 8] = accs[r8 % 8] + lax.dot_general(
            oh, ol, (((1,), (1,)), ((), ())),
            preferred_element_type=jnp.float32)

    o_ref[...] += ((accs[0] + accs[1]) + (accs[2] + accs[3])) + (
        (accs[4] + accs[5]) + (accs[6] + accs[7]))

    @pl.when(step == TC_NSTEPS - 1)
    def _final():
        cbig = o_ref[...]
        tot = jnp.zeros((16, 16), jnp.float32)
        for c in range(8):
            tot = tot + cbig[c * 16:(c + 1) * 16, c * 16:(c + 1) * 16]
        jj = lax.broadcasted_iota(jnp.int32, (16, 256), 1)
        ii = lax.broadcasted_iota(jnp.int32, (16, 256), 0)
        q = ((jj & 15) == ii).astype(jnp.float32)
        m2 = ((jj >> 4) == ii).astype(jnp.float32)
        z = jnp.dot(tot, q, preferred_element_type=jnp.float32) * m2
        h_ref[...] = jnp.dot(jnp.ones((1, 16), jnp.float32), z,
                             preferred_element_type=jnp.float32)


def _tc_hist(x2):
    return pl.pallas_call(
        _tc_hist_body,
        grid=(TC_NSTEPS,),
        in_specs=[pl.BlockSpec((TC_RB, 128), lambda i: (i + TC_BLOCK0, 0))],
        out_specs=[pl.BlockSpec((128, 128), lambda i: (0, 0)),
                   pl.BlockSpec((1, 256), lambda i: (0, 0))],
        out_shape=[jax.ShapeDtypeStruct((128, 128), jnp.float32),
                   jax.ShapeDtypeStruct((1, 256), jnp.float32)],
    )(x2)[1]


def _mi_body(p_ref, hxt_ref, hyt_ref, o_ref):
    p = p_ref[...]
    s = jnp.sum(p, axis=0, keepdims=True)
    hx = s[:, :BINS] + hxt_ref[...]
    hy = s[:, BINS:] + hyt_ref[...]
    sx = jnp.sum(hx)
    sy = jnp.sum(hy)
    jp = (hx + hy) / (sx + sy)
    px = hx / sx
    py = hy / sy
    ljp = jnp.log(jp)
    lpx = jnp.log(px)
    lpy = jnp.log(py)
    a = jp * (ljp - lpy)
    mi = float(BINS) * jnp.sum(a) - jnp.sum(lpx) * jnp.sum(jp)
    o_ref[...] = jnp.reshape(-mi, (1, 1))


def _mi_reduce(partials, hxt, hyt):
    return pl.pallas_call(
        _mi_body,
        out_shape=jax.ShapeDtypeStruct((1, 1), jnp.float32),
    )(partials, hxt, hyt)


@jax.jit
def kernel(x, y):
    partials = _make_hist_kernel()(x, y)
    hxt = _tc_hist(x.reshape(-1, 128))
    hyt = _tc_hist(y.reshape(-1, 128))
    out = _mi_reduce(partials, hxt, hyt)
    return out[0, 0]


# speedup vs baseline: 1.4227x; 1.1262x over previous
import jax
import jax.numpy as jnp
from jax import lax
from jax.experimental import pallas as pl
from jax.experimental.pallas import tpu as pltpu
from jax.experimental.pallas import tpu_sc as plsc

N = 16777216
NUM_WORKERS = 32
CHUNK = 16384
NCHUNKS = 22
PER_WORKER = NCHUNKS * CHUNK
A_SC = NUM_WORKERS * PER_WORKER
TC_RB = 512
TC_BLOCK0 = A_SC // (TC_RB * 128)
TC_NSTEPS = (N - A_SC) // (TC_RB * 128)
BINS = 256
DUMP = 512
REGION = 528
SHARED_N = 16 * REGION
NVST = 8
NSTR = 8
STR_WORDS = 64 * NSTR * 16


def _hist_body(x_hbm, y_hbm, out_hbm, buf0, buf1, hist, idx0, idx1, idx2,
               idx3, ones2d, zbuf, local, shared, sem0, sem1, ssem0, ssem1,
               ssem2, ssem3):
    core = lax.axis_index("c")
    sub = lax.axis_index("s")
    wid = sub * 2 + core
    base = wid * PER_WORKER
    sems = (sem0, sem1)
    ssems = (ssem0, ssem1, ssem2, ssem3)
    bufs_all = (buf0, buf1)
    idxbufs = (idx0, idx1, idx2, idx3)

    zeros16 = jnp.zeros((16,), jnp.float32)
    ones16 = jnp.ones((16,), jnp.float32)
    lane = lax.broadcasted_iota(jnp.int32, (16,), 0)

    def zb(i, _):
        hist[pl.ds(i * 16, 16)] = zeros16
        return 0

    lax.fori_loop(0, 512, zb, 0)

    def of(r, _):
        ones2d[pl.ds(r * 16, 16)] = ones16
        return 0

    lax.fori_loop(0, STR_WORDS // 16, of, 0)

    @pl.when(sub == 0)
    def _zero_shared():
        def zs(i, _):
            zbuf[pl.ds(i * 16, 16)] = zeros16
            return 0

        lax.fori_loop(0, SHARED_N // 16, zs, 0)
        pltpu.sync_copy(zbuf, shared)

    plsc.subcore_barrier()

    for which, src in enumerate((x_hbm, y_hbm)):
        lanewhich = lane + which * 4096
        woff = which * BINS + sub * REGION
        dump = sub * REGION + DUMP

        pltpu.async_copy(src.at[pl.ds(base, CHUNK)], buf0, sem0)

        def obody(g, _, src=src, lanewhich=lanewhich, woff=woff, dump=dump):
            for s2 in range(2):
                c = g * 2 + s2
                db = s2
                pltpu.make_async_copy(
                    src.at[pl.ds(base, CHUNK)], bufs_all[db], sems[db]
                ).wait()

                @pl.when(c + 1 < NCHUNKS)
                def _start(db=db, c=c, src=src):
                    pltpu.async_copy(
                        src.at[pl.ds(base + (c + 1) * CHUNK, CHUNK)],
                        bufs_all[db ^ 1],
                        sems[db ^ 1],
                    )

                @pl.when(g >= 1)
                def _wait_stream(s2=s2):
                    pltpu.make_async_copy(
                        ones2d, shared.at[idxbufs[s2]], ssems[s2]
                    ).wait()

                bufs = bufs_all[db]
                ib = idxbufs[s2]

                def vbody(i, _, bufs=bufs, ib=ib, lanewhich=lanewhich,
                          woff=woff, dump=dump):
                    b0 = i * 16
                    for k in range(NVST):
                        v = bufs[pl.ds((b0 + k) * 16, 16)]
                        t = (v + 4.0) * 32.0
                        u = t.astype(jnp.int32)
                        mask = (plsc.bitcast(t, jnp.uint32)
                                <= jnp.uint32(0x43800000))
                        idx = jnp.minimum(u, BINS - 1)
                        flat = (idx << 4) + lanewhich
                        plsc.addupdate_scatter(hist, [flat], ones16,
                                               mask=mask)
                    for k in range(NSTR):
                        v = bufs[pl.ds((b0 + NVST + k) * 16, 16)]
                        t = (v + 4.0) * 32.0
                        u = t.astype(jnp.int32)
                        mask = (plsc.bitcast(t, jnp.uint32)
                                <= jnp.uint32(0x43800000))
                        idx = jnp.minimum(u, BINS - 1)
                        sidx = jnp.where(mask, idx + woff, dump)
                        ib[pl.ds(i * (NSTR * 16) + k * 16, 16)] = sidx
                    return 0

                lax.fori_loop(0, 64, vbody, 0)

                pltpu.async_copy(ones2d, shared.at[ib], ssems[s2], add=True)
            return 0

        lax.fori_loop(0, NCHUNKS // 2, obody, 0)

        for p in range(2):
            pltpu.make_async_copy(
                ones2d, shared.at[idxbufs[p]], ssems[p]
            ).wait()

    for which in range(2):
        base_vec = lane * 16 + which * 4096
        for j in range(BINS // 16):
            acc = zeros16
            for l in range(16):
                acc = acc + plsc.load_gather(hist, [base_vec + (j * 256 + l)])
            local[pl.ds(which * 256 + j * 16, 16)] = acc

    plsc.subcore_barrier()

    @pl.when(sub == 0)
    def _merge():
        pltpu.sync_copy(shared, zbuf)
        for r in range(16):
            for j in range(512 // 16):
                lv = (local[pl.ds(j * 16, 16)]
                      + zbuf[pl.ds(r * REGION + j * 16, 16)])
                local[pl.ds(j * 16, 16)] = lv

    pltpu.sync_copy(local, out_hbm.at[wid])


def _make_hist_kernel():
    mesh = plsc.VectorSubcoreMesh(core_axis_name="c", subcore_axis_name="s")
    return pl.kernel(
        _hist_body,
        mesh=mesh,
        compiler_params=pltpu.CompilerParams(needs_layout_passes=False),
        out_type=jax.ShapeDtypeStruct((NUM_WORKERS, 512), jnp.float32),
        scratch_types=[
            pltpu.VMEM((CHUNK,), jnp.float32),
            pltpu.VMEM((CHUNK,), jnp.float32),
            pltpu.VMEM((8192,), jnp.float32),
            pltpu.VMEM((STR_WORDS,), jnp.int32),
            pltpu.VMEM((STR_WORDS,), jnp.int32),
            pltpu.VMEM((STR_WORDS,), jnp.int32),
            pltpu.VMEM((STR_WORDS,), jnp.int32),
            pltpu.VMEM((STR_WORDS,), jnp.float32),
            pltpu.VMEM((SHARED_N,), jnp.float32),
            pltpu.VMEM((512,), jnp.float32),
            pltpu.VMEM_SHARED((SHARED_N,), jnp.float32),
            pltpu.SemaphoreType.DMA,
            pltpu.SemaphoreType.DMA,
            pltpu.SemaphoreType.DMA,
            pltpu.SemaphoreType.DMA,
            pltpu.SemaphoreType.DMA,
            pltpu.SemaphoreType.DMA,
        ],
    )


def _tc_hist_body(x_ref, o_ref, h_ref):
    step = pl.program_id(0)

    @pl.when(step == 0)
    def _init():
        o_ref[...] = jnp.zeros((128, 128), jnp.float32)

    v = x_ref[...]
    t = (v + 4.0) * 32.0
    u = t.astype(jnp.int32)
    valid = lax.bitcast_convert_type(t, jnp.uint32) <= jnp.uint32(0x43800000)
    idx = jnp.minimum(u, 255)
    hi = jnp.where(valid, idx >> 4, 16)
    lo = idx & 15
    hibf = hi.astype(jnp.bfloat16)
    lobf = lo.astype(jnp.bfloat16)

    i128 = lax.broadcasted_iota(jnp.int32, (128, 1), 0)
    patc = (i128 & 15).astype(jnp.float32)
    rep = ((lax.broadcasted_iota(jnp.int32, (128, 8), 0) >> 4)
           == lax.broadcasted_iota(jnp.int32, (128, 8), 1)
           ).astype(jnp.bfloat16)

    accs = [jnp.zeros((128, 128), jnp.float32) for _ in range(8)]
    for r8 in range(TC_RB // 8):
        h8 = hibf[r8 * 8:(r8 + 1) * 8, :]
        l8 = lobf[r8 * 8:(r8 + 1) * 8, :]
        eh = jnp.dot(rep, h8, preferred_element_type=jnp.float32)
        el = jnp.dot(rep, l8, preferred_element_type=jnp.float32)
        oh = (eh == patc).astype(jnp.bfloat16)
        ol = (el == patc).astype(jnp.bfloat16)
        accs[r8 % 8] = accs[r8 % 8] + lax.dot_general(
            oh, ol, (((1,), (1,)), ((), ())),
            preferred_element_type=jnp.float32)

    o_ref[...] += ((accs[0] + accs[1]) + (accs[2] + accs[3])) + (
        (accs[4] + accs[5]) + (accs[6] + accs[7]))

    @pl.when(step == TC_NSTEPS - 1)
    def _final():
        cbig = o_ref[...]
        tot = jnp.zeros((16, 16), jnp.float32)
        for c in range(8):
            tot = tot + cbig[c * 16:(c + 1) * 16, c * 16:(c + 1) * 16]
        jj = lax.broadcasted_iota(jnp.int32, (16, 256), 1)
        ii = lax.broadcasted_iota(jnp.int32, (16, 256), 0)
        q = ((jj & 15) == ii).astype(jnp.float32)
        m2 = ((jj >> 4) == ii).astype(jnp.float32)
        z = jnp.dot(tot, q, preferred_element_type=jnp.float32) * m2
        h_ref[...] = jnp.dot(jnp.ones((1, 16), jnp.float32), z,
                             preferred_element_type=jnp.float32)


def _tc_hist(x2):
    return pl.pallas_call(
        _tc_hist_body,
        grid=(TC_NSTEPS,),
        in_specs=[pl.BlockSpec((TC_RB, 128), lambda i: (i + TC_BLOCK0, 0))],
        out_specs=[pl.BlockSpec((128, 128), lambda i: (0, 0)),
                   pl.BlockSpec((1, 256), lambda i: (0, 0))],
        out_shape=[jax.ShapeDtypeStruct((128, 128), jnp.float32),
                   jax.ShapeDtypeStruct((1, 256), jnp.float32)],
    )(x2)[1]


def _mi_body(p_ref, hxt_ref, hyt_ref, o_ref):
    p = p_ref[...]
    s = jnp.sum(p, axis=0, keepdims=True)
    hx = s[:, :BINS] + hxt_ref[...]
    hy = s[:, BINS:] + hyt_ref[...]
    sx = jnp.sum(hx)
    sy = jnp.sum(hy)
    jp = (hx + hy) / (sx + sy)
    px = hx / sx
    py = hy / sy
    ljp = jnp.log(jp)
    lpx = jnp.log(px)
    lpy = jnp.log(py)
    a = jp * (ljp - lpy)
    mi = float(BINS) * jnp.sum(a) - jnp.sum(lpx) * jnp.sum(jp)
    o_ref[...] = jnp.reshape(-mi, (1, 1))


def _mi_reduce(partials, hxt, hyt):
    return pl.pallas_call(
        _mi_body,
        out_shape=jax.ShapeDtypeStruct((1, 1), jnp.float32),
    )(partials, hxt, hyt)


@jax.jit
def kernel(x, y):
    partials = _make_hist_kernel()(x, y)
    hxt = _tc_hist(x.reshape(-1, 128))
    hyt = _tc_hist(y.reshape(-1, 128))
    out = _mi_reduce(partials, hxt, hyt)
    return out[0, 0]


# speedup vs baseline: 1.4589x; 1.0255x over previous
import jax
import jax.numpy as jnp
from jax import lax
from jax.experimental import pallas as pl
from jax.experimental.pallas import tpu as pltpu
from jax.experimental.pallas import tpu_sc as plsc

N = 16777216
NUM_WORKERS = 32
CHUNK = 16384
NCHUNKS = 22
PER_WORKER = NCHUNKS * CHUNK
A_SC = NUM_WORKERS * PER_WORKER
TC_RB = 512
TC_BLOCK0 = A_SC // (TC_RB * 128)
TC_NSTEPS = (N - A_SC) // (TC_RB * 128)
BINS = 256
DUMP = 512
REGION = 528
SHARED_N = 16 * REGION
NVST = 6
NSTR = 10
STR_WORDS = 64 * NSTR * 16


def _hist_body(x_hbm, y_hbm, out_hbm, buf0, buf1, hist, idx0, idx1, idx2,
               idx3, ones2d, zbuf, local, shared, sem0, sem1, ssem0, ssem1,
               ssem2, ssem3):
    core = lax.axis_index("c")
    sub = lax.axis_index("s")
    wid = sub * 2 + core
    base = wid * PER_WORKER
    sems = (sem0, sem1)
    ssems = (ssem0, ssem1, ssem2, ssem3)
    bufs_all = (buf0, buf1)
    idxbufs = (idx0, idx1, idx2, idx3)

    zeros16 = jnp.zeros((16,), jnp.float32)
    ones16 = jnp.ones((16,), jnp.float32)
    lane = lax.broadcasted_iota(jnp.int32, (16,), 0)

    def zb(i, _):
        hist[pl.ds(i * 16, 16)] = zeros16
        return 0

    lax.fori_loop(0, 512, zb, 0)

    def of(r, _):
        ones2d[pl.ds(r * 16, 16)] = ones16
        return 0

    lax.fori_loop(0, STR_WORDS // 16, of, 0)

    @pl.when(sub == 0)
    def _zero_shared():
        def zs(i, _):
            zbuf[pl.ds(i * 16, 16)] = zeros16
            return 0

        lax.fori_loop(0, SHARED_N // 16, zs, 0)
        pltpu.sync_copy(zbuf, shared)

    plsc.subcore_barrier()

    for which, src in enumerate((x_hbm, y_hbm)):
        lanewhich = lane + which * 4096
        woff = which * BINS + sub * REGION
        dump = sub * REGION + DUMP

        pltpu.async_copy(src.at[pl.ds(base, CHUNK)], buf0, sem0)

        def obody(g, _, src=src, lanewhich=lanewhich, woff=woff, dump=dump):
            for s2 in range(2):
                c = g * 2 + s2
                db = s2
                pltpu.make_async_copy(
                    src.at[pl.ds(base, CHUNK)], bufs_all[db], sems[db]
                ).wait()

                @pl.when(c + 1 < NCHUNKS)
                def _start(db=db, c=c, src=src):
                    pltpu.async_copy(
                        src.at[pl.ds(base + (c + 1) * CHUNK, CHUNK)],
                        bufs_all[db ^ 1],
                        sems[db ^ 1],
                    )

                @pl.when(g >= 1)
                def _wait_stream(s2=s2):
                    pltpu.make_async_copy(
                        ones2d, shared.at[idxbufs[s2]], ssems[s2]
                    ).wait()

                bufs = bufs_all[db]
                ib = idxbufs[s2]

                def vbody(i, _, bufs=bufs, ib=ib, lanewhich=lanewhich,
                          woff=woff, dump=dump):
                    b0 = i * 16
                    for k in range(NVST):
                        v = bufs[pl.ds((b0 + k) * 16, 16)]
                        t = (v + 4.0) * 32.0
                        u = t.astype(jnp.int32)
                        mask = (plsc.bitcast(t, jnp.uint32)
                                <= jnp.uint32(0x43800000))
                        idx = jnp.minimum(u, BINS - 1)
                        flat = (idx << 4) + lanewhich
                        plsc.addupdate_scatter(hist, [flat], ones16,
                                               mask=mask)
                    for k in range(NSTR):
                        v = bufs[pl.ds((b0 + NVST + k) * 16, 16)]
                        t = (v + 4.0) * 32.0
                        u = t.astype(jnp.int32)
                        mask = (plsc.bitcast(t, jnp.uint32)
                                <= jnp.uint32(0x43800000))
                        idx = jnp.minimum(u, BINS - 1)
                        sidx = jnp.where(mask, idx + woff, dump)
                        ib[pl.ds(i * (NSTR * 16) + k * 16, 16)] = sidx
                    return 0

                lax.fori_loop(0, 64, vbody, 0)

                pltpu.async_copy(ones2d, shared.at[ib], ssems[s2], add=True)
            return 0

        lax.fori_loop(0, NCHUNKS // 2, obody, 0)

        for p in range(2):
            pltpu.make_async_copy(
                ones2d, shared.at[idxbufs[p]], ssems[p]
            ).wait()

    for which in range(2):
        base_vec = lane * 16 + which * 4096
        for j in range(BINS // 16):
            acc = zeros16
            for l in range(16):
                acc = acc + plsc.load_gather(hist, [base_vec + (j * 256 + l)])
            local[pl.ds(which * 256 + j * 16, 16)] = acc

    plsc.subcore_barrier()

    @pl.when(sub == 0)
    def _merge():
        pltpu.sync_copy(shared, zbuf)
        for r in range(16):
            for j in range(512 // 16):
                lv = (local[pl.ds(j * 16, 16)]
                      + zbuf[pl.ds(r * REGION + j * 16, 16)])
                local[pl.ds(j * 16, 16)] = lv

    pltpu.sync_copy(local, out_hbm.at[wid])


def _make_hist_kernel():
    mesh = plsc.VectorSubcoreMesh(core_axis_name="c", subcore_axis_name="s")
    return pl.kernel(
        _hist_body,
        mesh=mesh,
        compiler_params=pltpu.CompilerParams(needs_layout_passes=False),
        out_type=jax.ShapeDtypeStruct((NUM_WORKERS, 512), jnp.float32),
        scratch_types=[
            pltpu.VMEM((CHUNK,), jnp.float32),
            pltpu.VMEM((CHUNK,), jnp.float32),
            pltpu.VMEM((8192,), jnp.float32),
            pltpu.VMEM((STR_WORDS,), jnp.int32),
            pltpu.VMEM((STR_WORDS,), jnp.int32),
            pltpu.VMEM((STR_WORDS,), jnp.int32),
            pltpu.VMEM((STR_WORDS,), jnp.int32),
            pltpu.VMEM((STR_WORDS,), jnp.float32),
            pltpu.VMEM((SHARED_N,), jnp.float32),
            pltpu.VMEM((512,), jnp.float32),
            pltpu.VMEM_SHARED((SHARED_N,), jnp.float32),
            pltpu.SemaphoreType.DMA,
            pltpu.SemaphoreType.DMA,
            pltpu.SemaphoreType.DMA,
            pltpu.SemaphoreType.DMA,
            pltpu.SemaphoreType.DMA,
            pltpu.SemaphoreType.DMA,
        ],
    )


def _tc_hist_body(x_ref, o_ref, h_ref):
    step = pl.program_id(0)

    @pl.when(step == 0)
    def _init():
        o_ref[...] = jnp.zeros((128, 128), jnp.float32)

    v = x_ref[...]
    t = (v + 4.0) * 32.0
    u = t.astype(jnp.int32)
    valid = lax.bitcast_convert_type(t, jnp.uint32) <= jnp.uint32(0x43800000)
    idx = jnp.minimum(u, 255)
    hi = jnp.where(valid, idx >> 4, 16)
    lo = idx & 15
    hibf = hi.astype(jnp.bfloat16)
    lobf = lo.astype(jnp.bfloat16)

    i128 = lax.broadcasted_iota(jnp.int32, (128, 1), 0)
    patc = (i128 & 15).astype(jnp.float32)
    rep = ((lax.broadcasted_iota(jnp.int32, (128, 8), 0) >> 4)
           == lax.broadcasted_iota(jnp.int32, (128, 8), 1)
           ).astype(jnp.bfloat16)

    accs = [jnp.zeros((128, 128), jnp.float32) for _ in range(8)]
    for r8 in range(TC_RB // 8):
        h8 = hibf[r8 * 8:(r8 + 1) * 8, :]
        l8 = lobf[r8 * 8:(r8 + 1) * 8, :]
        eh = jnp.dot(rep, h8, preferred_element_type=jnp.float32)
        el = jnp.dot(rep, l8, preferred_element_type=jnp.float32)
        oh = (eh == patc).astype(jnp.bfloat16)
        ol = (el == patc).astype(jnp.bfloat16)
        accs[r8 % 8] = accs[r8 % 8] + lax.dot_general(
            oh, ol, (((1,), (1,)), ((), ())),
            preferred_element_type=jnp.float32)

    o_ref[...] += ((accs[0] + accs[1]) + (accs[2] + accs[3])) + (
        (accs[4] + accs[5]) + (accs[6] + accs[7]))

    @pl.when(step == TC_NSTEPS - 1)
    def _final():
        cbig = o_ref[...]
        tot = jnp.zeros((16, 16), jnp.float32)
        for c in range(8):
            tot = tot + cbig[c * 16:(c + 1) * 16, c * 16:(c + 1) * 16]
        jj = lax.broadcasted_iota(jnp.int32, (16, 256), 1)
        ii = lax.broadcasted_iota(jnp.int32, (16, 256), 0)
        q = ((jj & 15) == ii).astype(jnp.float32)
        m2 = ((jj >> 4) == ii).astype(jnp.float32)
        z = jnp.dot(tot, q, preferred_element_type=jnp.float32) * m2
        h_ref[...] = jnp.dot(jnp.ones((1, 16), jnp.float32), z,
                             preferred_element_type=jnp.float32)


def _tc_hist(x2):
    return pl.pallas_call(
        _tc_hist_body,
        grid=(TC_NSTEPS,),
        in_specs=[pl.BlockSpec((TC_RB, 128), lambda i: (i + TC_BLOCK0, 0))],
        out_specs=[pl.BlockSpec((128, 128), lambda i: (0, 0)),
                   pl.BlockSpec((1, 256), lambda i: (0, 0))],
        out_shape=[jax.ShapeDtypeStruct((128, 128), jnp.float32),
                   jax.ShapeDtypeStruct((1, 256), jnp.float32)],
    )(x2)[1]


def _mi_body(p_ref, hxt_ref, hyt_ref, o_ref):
    p = p_ref[...]
    s = jnp.sum(p, axis=0, keepdims=True)
    hx = s[:, :BINS] + hxt_ref[...]
    hy = s[:, BINS:] + hyt_ref[...]
    sx = jnp.sum(hx)
    sy = jnp.sum(hy)
    jp = (hx + hy) / (sx + sy)
    px = hx / sx
    py = hy / sy
    ljp = jnp.log(jp)
    lpx = jnp.log(px)
    lpy = jnp.log(py)
    a = jp * (ljp - lpy)
    mi = float(BINS) * jnp.sum(a) - jnp.sum(lpx) * jnp.sum(jp)
    o_ref[...] = jnp.reshape(-mi, (1, 1))


def _mi_reduce(partials, hxt, hyt):
    return pl.pallas_call(
        _mi_body,
        out_shape=jax.ShapeDtypeStruct((1, 1), jnp.float32),
    )(partials, hxt, hyt)


@jax.jit
def kernel(x, y):
    partials = _make_hist_kernel()(x, y)
    hxt = _tc_hist(x.reshape(-1, 128))
    hyt = _tc_hist(y.reshape(-1, 128))
    out = _mi_reduce(partials, hxt, hyt)
    return out[0, 0]


# speedup vs baseline: 1.4589x; 1.0000x over previous
import jax
import jax.numpy as jnp
from jax import lax
from jax.experimental import pallas as pl
from jax.experimental.pallas import tpu as pltpu
from jax.experimental.pallas import tpu_sc as plsc

N = 16777216
NUM_WORKERS = 32
CHUNK = 16384
NCHUNKS = 22
PER_WORKER = NCHUNKS * CHUNK
A_SC = NUM_WORKERS * PER_WORKER
TC_RB = 512
TC_BLOCK0 = A_SC // (TC_RB * 128)
TC_NSTEPS = (N - A_SC) // (TC_RB * 128)
BINS = 256
DUMP = 512
REGION = 528
SHARED_N = 16 * REGION
NVST = 6
NSTR = 10
STR_WORDS = 64 * NSTR * 16


def _hist_body(x_hbm, y_hbm, out_hbm, buf0, buf1, hist, idx0, idx1, ones2d,
               zbuf, local, shared, sem0, sem1, ssem0, ssem1):
    core = lax.axis_index("c")
    sub = lax.axis_index("s")
    wid = sub * 2 + core
    base = wid * PER_WORKER
    sems = (sem0, sem1)
    ssems = (ssem0, ssem1)
    bufs_all = (buf0, buf1)
    idxbufs = (idx0, idx1)

    zeros16 = jnp.zeros((16,), jnp.float32)
    ones16 = jnp.ones((16,), jnp.float32)
    lane = lax.broadcasted_iota(jnp.int32, (16,), 0)

    def zb(i, _):
        hist[pl.ds(i * 16, 16)] = zeros16
        return 0

    lax.fori_loop(0, 512, zb, 0)

    def of(r, _):
        ones2d[pl.ds(r * 16, 16)] = ones16
        return 0

    lax.fori_loop(0, STR_WORDS // 16, of, 0)

    @pl.when(sub == 0)
    def _zero_shared():
        def zs(i, _):
            zbuf[pl.ds(i * 16, 16)] = zeros16
            return 0

        lax.fori_loop(0, SHARED_N // 16, zs, 0)
        pltpu.sync_copy(zbuf, shared)

    plsc.subcore_barrier()

    for which, src in enumerate((x_hbm, y_hbm)):
        lanewhich = lane + which * 4096
        woff = which * BINS + sub * REGION
        dump = sub * REGION + DUMP

        pltpu.async_copy(src.at[pl.ds(base, CHUNK)], buf0, sem0)

        def obody(g, _, src=src, lanewhich=lanewhich, woff=woff, dump=dump):
            for s2 in range(2):
                c = g * 2 + s2
                db = s2
                pltpu.make_async_copy(
                    src.at[pl.ds(base, CHUNK)], bufs_all[db], sems[db]
                ).wait()

                @pl.when(c + 1 < NCHUNKS)
                def _start(db=db, c=c, src=src):
                    pltpu.async_copy(
                        src.at[pl.ds(base + (c + 1) * CHUNK, CHUNK)],
                        bufs_all[db ^ 1],
                        sems[db ^ 1],
                    )

                @pl.when(g >= 1)
                def _wait_stream(s2=s2):
                    pltpu.make_async_copy(
                        ones2d, shared.at[idxbufs[s2]], ssems[s2]
                    ).wait()

                bufs = bufs_all[db]
                ib = idxbufs[s2]

                def vbody(i, _, bufs=bufs, ib=ib, lanewhich=lanewhich,
                          woff=woff, dump=dump):
                    b0 = i * 16
                    for k in range(NVST):
                        v = bufs[pl.ds((b0 + k) * 16, 16)]
                        t = (v + 4.0) * 32.0
                        u = t.astype(jnp.int32)
                        mask = (plsc.bitcast(t, jnp.uint32)
                                <= jnp.uint32(0x43800000))
                        idx = jnp.minimum(u, BINS - 1)
                        flat = (idx << 4) + lanewhich
                        plsc.addupdate_scatter(hist, [flat], ones16,
                                               mask=mask)
                    for k in range(NSTR):
                        v = bufs[pl.ds((b0 + NVST + k) * 16, 16)]
                        t = (v + 4.0) * 32.0
                        u = t.astype(jnp.int32)
                        mask = (plsc.bitcast(t, jnp.uint32)
                                <= jnp.uint32(0x43800000))
                        idx = jnp.minimum(u, BINS - 1)
                        sidx = jnp.where(mask, idx + woff, dump)
                        ib[pl.ds(i * (NSTR * 16) + k * 16, 16)] = sidx
                    return 0

                lax.fori_loop(0, 64, vbody, 0)

                pltpu.async_copy(ones2d, shared.at[ib], ssems[s2], add=True)
            return 0

        lax.fori_loop(0, NCHUNKS // 2, obody, 0)

        for p in range(2):
            pltpu.make_async_copy(
                ones2d, shared.at[idxbufs[p]], ssems[p]
            ).wait()

    for which in range(2):
        base_vec = lane * 16 + which * 4096
        for j in range(BINS // 16):
            acc = zeros16
            for l in range(16):
                acc = acc + plsc.load_gather(hist, [base_vec + (j * 256 + l)])
            local[pl.ds(which * 256 + j * 16, 16)] = acc

    plsc.subcore_barrier()

    @pl.when(sub == 0)
    def _merge():
        pltpu.sync_copy(shared, zbuf)
        for r in range(16):
            for j in range(512 // 16):
                lv = (local[pl.ds(j * 16, 16)]
                      + zbuf[pl.ds(r * REGION + j * 16, 16)])
                local[pl.ds(j * 16, 16)] = lv

    pltpu.sync_copy(local, out_hbm.at[wid])


def _make_hist_kernel():
    mesh = plsc.VectorSubcoreMesh(core_axis_name="c", subcore_axis_name="s")
    return pl.kernel(
        _hist_body,
        mesh=mesh,
        compiler_params=pltpu.CompilerParams(needs_layout_passes=False),
        out_type=jax.ShapeDtypeStruct((NUM_WORKERS, 512), jnp.float32),
        scratch_types=[
            pltpu.VMEM((CHUNK,), jnp.float32),
            pltpu.VMEM((CHUNK,), jnp.float32),
            pltpu.VMEM((8192,), jnp.float32),
            pltpu.VMEM((STR_WORDS,), jnp.int32),
            pltpu.VMEM((STR_WORDS,), jnp.int32),
            pltpu.VMEM((STR_WORDS,), jnp.float32),
            pltpu.VMEM((SHARED_N,), jnp.float32),
            pltpu.VMEM((512,), jnp.float32),
            pltpu.VMEM_SHARED((SHARED_N,), jnp.float32),
            pltpu.SemaphoreType.DMA,
            pltpu.SemaphoreType.DMA,
            pltpu.SemaphoreType.DMA,
            pltpu.SemaphoreType.DMA,
        ],
    )


def _tc_hist_body(x_ref, o_ref, h_ref):
    step = pl.program_id(0)

    @pl.when(step == 0)
    def _init():
        o_ref[...] = jnp.zeros((128, 128), jnp.float32)

    v = x_ref[...]
    t = (v + 4.0) * 32.0
    u = t.astype(jnp.int32)
    valid = lax.bitcast_convert_type(t, jnp.uint32) <= jnp.uint32(0x43800000)
    idx = jnp.minimum(u, 255)
    hi = jnp.where(valid, idx >> 4, 16)
    lo = idx & 15
    hibf = hi.astype(jnp.bfloat16)
    lobf = lo.astype(jnp.bfloat16)

    i128 = lax.broadcasted_iota(jnp.int32, (128, 1), 0)
    patc = (i128 & 15).astype(jnp.float32)
    rep = ((lax.broadcasted_iota(jnp.int32, (128, 8), 0) >> 4)
           == lax.broadcasted_iota(jnp.int32, (128, 8), 1)
           ).astype(jnp.bfloat16)

    accs = [jnp.zeros((128, 128), jnp.float32) for _ in range(8)]
    for r8 in range(TC_RB // 8):
        h8 = hibf[r8 * 8:(r8 + 1) * 8, :]
        l8 = lobf[r8 * 8:(r8 + 1) * 8, :]
        eh = jnp.dot(rep, h8, preferred_element_type=jnp.float32)
        el = jnp.dot(rep, l8, preferred_element_type=jnp.float32)
        oh = (eh == patc).astype(jnp.bfloat16)
        ol = (el == patc).astype(jnp.bfloat16)
        accs[r8 % 8] = accs[r8 % 8] + lax.dot_general(
            oh, ol, (((1,), (1,)), ((), ())),
            preferred_element_type=jnp.float32)

    o_ref[...] += ((accs[0] + accs[1]) + (accs[2] + accs[3])) + (
        (accs[4] + accs[5]) + (accs[6] + accs[7]))

    @pl.when(step == TC_NSTEPS - 1)
    def _final():
        cbig = o_ref[...]
        tot = jnp.zeros((16, 16), jnp.float32)
        for c in range(8):
            tot = tot + cbig[c * 16:(c + 1) * 16, c * 16:(c + 1) * 16]
        jj = lax.broadcasted_iota(jnp.int32, (16, 256), 1)
        ii = lax.broadcasted_iota(jnp.int32, (16, 256), 0)
        q = ((jj & 15) == ii).astype(jnp.float32)
        m2 = ((jj >> 4) == ii).astype(jnp.float32)
        z = jnp.dot(tot, q, preferred_element_type=jnp.float32) * m2
        h_ref[...] = jnp.dot(jnp.ones((1, 16), jnp.float32), z,
                             preferred_element_type=jnp.float32)


def _tc_hist(x2):
    return pl.pallas_call(
        _tc_hist_body,
        grid=(TC_NSTEPS,),
        in_specs=[pl.BlockSpec((TC_RB, 128), lambda i: (i + TC_BLOCK0, 0))],
        out_specs=[pl.BlockSpec((128, 128), lambda i: (0, 0)),
                   pl.BlockSpec((1, 256), lambda i: (0, 0))],
        out_shape=[jax.ShapeDtypeStruct((128, 128), jnp.float32),
                   jax.ShapeDtypeStruct((1, 256), jnp.float32)],
    )(x2)[1]


def _mi_body(p_ref, hxt_ref, hyt_ref, o_ref):
    p = p_ref[...]
    s = jnp.sum(p, axis=0, keepdims=True)
    hx = s[:, :BINS] + hxt_ref[...]
    hy = s[:, BINS:] + hyt_ref[...]
    sx = jnp.sum(hx)
    sy = jnp.sum(hy)
    jp = (hx + hy) / (sx + sy)
    px = hx / sx
    py = hy / sy
    ljp = jnp.log(jp)
    lpx = jnp.log(px)
    lpy = jnp.log(py)
    a = jp * (ljp - lpy)
    mi = float(BINS) * jnp.sum(a) - jnp.sum(lpx) * jnp.sum(jp)
    o_ref[...] = jnp.reshape(-mi, (1, 1))


def _mi_reduce(partials, hxt, hyt):
    return pl.pallas_call(
        _mi_body,
        out_shape=jax.ShapeDtypeStruct((1, 1), jnp.float32),
    )(partials, hxt, hyt)


@jax.jit
def kernel(x, y):
    partials = _make_hist_kernel()(x, y)
    hxt = _tc_hist(x.reshape(-1, 128))
    hyt = _tc_hist(y.reshape(-1, 128))
    out = _mi_reduce(partials, hxt, hyt)
    return out[0, 0]
